# Initial kernel scaffold; baseline (speedup 1.0000x reference)
import jax
import jax.numpy as jnp
from jax.experimental import pallas as pl


def kernel(x, edge_index, W1, a1s, a1d, b1, W2, a2s, a2d, b2, W3, a3s, a3d, b3):
    raise NotImplementedError("write your pallas kernel here")


# speedup vs baseline: 19.5072x; 19.5072x over previous
import jax
import jax.numpy as jnp
from jax import lax
from jax.experimental import pallas as pl
from jax.experimental.pallas import tpu as pltpu
from jax.experimental.pallas import tpu_sc as plsc

N = 10000
NPAD = 10240
E_RAW = 320000
E_TOT = E_RAW + N
BLK = 128
NC, NS = 2, 16
NW = NC * NS
NBT = -(-E_TOT // BLK)
NBT = -(-NBT // NW) * NW
E_PAD = NBT * BLK
NB_W = NBT // NW
NB_C = NBT // NS
ZR = NPAD // NS
H = 6
RB = 4096
GE = E_PAD // RB
F32 = jnp.float32


def _mesh():
    return plsc.VectorSubcoreMesh(core_axis_name="c", subcore_axis_name="s")


_SC_PARAMS = pltpu.CompilerParams(use_tc_tiling_on_sc=False)


def _sc_gather(table, idx2, d):
    def body(table_hbm, idx_hbm, out_hbm, idx_v, rows_v, sem):
        wid = lax.axis_index("s") * NC + lax.axis_index("c")
        pltpu.sync_copy(idx_hbm.at[wid], idx_v)

        def step(j, carry):
            pltpu.async_copy(table_hbm.at[idx_v.at[j]], rows_v, sem).wait()
            pltpu.sync_copy(rows_v, out_hbm.at[pl.ds((wid * NB_W + j) * BLK, BLK)])
            return carry

        lax.fori_loop(0, NB_W, step, 0)

    return pl.kernel(
        body,
        out_type=jax.ShapeDtypeStruct((E_PAD, d), F32),
        mesh=_mesh(),
        scratch_types=[
            pltpu.VMEM((NB_W, BLK), jnp.int32),
            pltpu.VMEM((BLK, d), F32),
            pltpu.SemaphoreType.DMA,
        ],
        compiler_params=_SC_PARAMS,
    )(table, idx2)


def _sc_scatter(vals, idx2, zrows, d):
    def body(vals_hbm, idx_hbm, z_hbm, out_hbm, idx_v, vals_v, acc, sem):
        cid = lax.axis_index("c")
        sid = lax.axis_index("s")
        wid = sid * NC + cid
        pltpu.sync_copy(z_hbm, acc.at[pl.ds(sid * ZR, ZR)])
        pltpu.sync_copy(idx_hbm.at[wid], idx_v)
        plsc.subcore_barrier()

        def step(j, carry):
            pltpu.sync_copy(vals_hbm.at[pl.ds((wid * NB_W + j) * BLK, BLK)], vals_v)
            pltpu.sync_copy(vals_v, acc.at[idx_v.at[j]], add=True)
            return carry

        lax.fori_loop(0, NB_W, step, 0)
        plsc.subcore_barrier()
        pltpu.sync_copy(acc.at[pl.ds(sid * ZR, ZR)],
                        out_hbm.at[pl.ds(cid * NPAD + sid * ZR, ZR)])

    return pl.kernel(
        body,
        out_type=jax.ShapeDtypeStruct((NC * NPAD, d), F32),
        mesh=_mesh(),
        scratch_types=[
            pltpu.VMEM((NB_W, BLK), jnp.int32),
            pltpu.VMEM((BLK, d), F32),
            pltpu.VMEM_SHARED((NPAD, d), F32),
            pltpu.SemaphoreType.DMA,
        ],
        compiler_params=_SC_PARAMS,
    )(vals, idx2, zrows)


def _sc_scatter3(vals, idx2, zrows):
    d = 112

    def body(vals_hbm, idx_hbm, z_hbm, out_hbm, idx_v, vals_v, acc, sem):
        cid = lax.axis_index("c")
        sid = lax.axis_index("s")
        pltpu.sync_copy(z_hbm, acc.at[pl.ds(sid * ZR, ZR)])
        pltpu.sync_copy(idx_hbm.at[sid], idx_v)
        plsc.subcore_barrier()

        def step(j, carry):
            pltpu.sync_copy(
                vals_hbm.at[pl.ds((sid * NB_C + j) * BLK, BLK), pl.ds(cid * d, d)],
                vals_v)
            pltpu.sync_copy(vals_v, acc.at[idx_v.at[j]], add=True)
            return carry

        lax.fori_loop(0, NB_C, step, 0)
        plsc.subcore_barrier()
        pltpu.sync_copy(acc.at[pl.ds(sid * ZR, ZR)],
                        out_hbm.at[pl.ds(cid * NPAD + sid * ZR, ZR)])

    return pl.kernel(
        body,
        out_type=jax.ShapeDtypeStruct((NC * NPAD, d), F32),
        mesh=_mesh(),
        scratch_types=[
            pltpu.VMEM((NB_C, BLK), jnp.int32),
            pltpu.VMEM((BLK, d), F32),
            pltpu.VMEM_SHARED((NPAD, d), F32),
            pltpu.SemaphoreType.DMA,
        ],
        compiler_params=_SC_PARAMS,
    )(vals, idx2, zrows)


_R = 2560


def _tc_prep12(h, W, a_s, a_d, din):
    def body(h_ref, w_ref, as_ref, ad_ref, a_ref, b_ref):
        xw = jnp.dot(h_ref[...], w_ref[...], preferred_element_type=F32)
        x3 = xw.reshape(_R, H, 6)
        asr = jnp.sum(x3 * as_ref[...][None], axis=-1)
        ads = jnp.sum(x3 * ad_ref[...][None], axis=-1)
        z = jnp.zeros((_R, 6), F32)
        a_ref[...] = jnp.concatenate([xw, asr, z], axis=1)
        b_ref[...] = jnp.concatenate([ads, jnp.zeros((_R, 2), F32)], axis=1)

    return pl.pallas_call(
        body,
        grid=(NPAD // _R,),
        in_specs=[
            pl.BlockSpec((_R, din), lambda i: (i, 0)),
            pl.BlockSpec((din, H * 6), lambda i: (0, 0)),
            pl.BlockSpec((H, 6), lambda i: (0, 0)),
            pl.BlockSpec((H, 6), lambda i: (0, 0)),
        ],
        out_specs=[
            pl.BlockSpec((_R, 48), lambda i: (i, 0)),
            pl.BlockSpec((_R, 8), lambda i: (i, 0)),
        ],
        out_shape=[
            jax.ShapeDtypeStruct((NPAD, 48), F32),
            jax.ShapeDtypeStruct((NPAD, 8), F32),
        ],
    )(h, W, a_s, a_d)


def _tc_prep3(h, W, a_s, a_d):
    def body(h_ref, w_ref, as_ref, ad_ref, a_ref, b_ref):
        hv = h_ref[...]
        xw = jnp.dot(hv, w_ref[...], preferred_element_type=F32)
        x3 = xw.reshape(_R, H, 128)
        asr = jnp.sum(x3 * as_ref[...][None], axis=-1)
        ads = jnp.sum(x3 * ad_ref[...][None], axis=-1)
        z = jnp.zeros((_R, 6), F32)
        a_ref[...] = jnp.concatenate([hv, asr, z], axis=1)
        b_ref[...] = jnp.concatenate([ads, jnp.zeros((_R, 2), F32)], axis=1)

    return pl.pallas_call(
        body,
        grid=(NPAD // _R,),
        in_specs=[
            pl.BlockSpec((_R, 36), lambda i: (i, 0)),
            pl.BlockSpec((36, H * 128), lambda i: (0, 0)),
            pl.BlockSpec((H, 128), lambda i: (0, 0)),
            pl.BlockSpec((H, 128), lambda i: (0, 0)),
        ],
        out_specs=[
            pl.BlockSpec((_R, 48), lambda i: (i, 0)),
            pl.BlockSpec((_R, 8), lambda i: (i, 0)),
        ],
        out_shape=[
            jax.ShapeDtypeStruct((NPAD, 48), F32),
            jax.ShapeDtypeStruct((NPAD, 8), F32),
        ],
    )(h, W, a_s, a_d)


def _tc_ex(a_g, b_g):
    def body(a_ref, b_ref, o_ref):
        al = a_ref[...][:, 36:42] + b_ref[...][:, 0:6]
        al = jnp.maximum(al, 0.2 * al)
        o_ref[...] = jnp.concatenate(
            [jnp.exp(al), jnp.zeros((RB, 2), F32)], axis=1)

    return pl.pallas_call(
        body,
        grid=(GE,),
        in_specs=[
            pl.BlockSpec((RB, 48), lambda i: (i, 0)),
            pl.BlockSpec((RB, 8), lambda i: (i, 0)),
        ],
        out_specs=pl.BlockSpec((RB, 8), lambda i: (i, 0)),
        out_shape=jax.ShapeDtypeStruct((E_PAD, 8), F32),
    )(a_g, b_g)


def _tc_rdenom(p):
    def body(p_ref, o_ref):
        d = p_ref[0:NPAD, :] + p_ref[NPAD:2 * NPAD, :]
        o_ref[...] = 1.0 / (d + 1e-16)

    return pl.pallas_call(
        body,
        out_shape=jax.ShapeDtypeStruct((NPAD, 8), F32),
    )(p)


def _tc_msg12(a_g, ex8, rd_g):
    def body(a_ref, e_ref, r_ref, o_ref):
        coef = e_ref[...][:, 0:6] * r_ref[...][:, 0:6]
        xws = a_ref[...][:, 0:36].reshape(RB, H, 6)
        m = (xws * coef[:, :, None]).reshape(RB, 36)
        o_ref[...] = jnp.concatenate([m, jnp.zeros((RB, 12), F32)], axis=1)

    return pl.pallas_call(
        body,
        grid=(GE,),
        in_specs=[
            pl.BlockSpec((RB, 48), lambda i: (i, 0)),
            pl.BlockSpec((RB, 8), lambda i: (i, 0)),
            pl.BlockSpec((RB, 8), lambda i: (i, 0)),
        ],
        out_specs=pl.BlockSpec((RB, 48), lambda i: (i, 0)),
        out_shape=jax.ShapeDtypeStruct((E_PAD, 48), F32),
    )(a_g, ex8, rd_g)


def _tc_msg3(a_g, ex8, rd_g):
    def body(a_ref, e_ref, r_ref, o_ref):
        coef = e_ref[...][:, 0:6] * r_ref[...][:, 0:6]
        h2s = a_ref[...][:, 0:36]
        m = h2s[:, None, :] * coef[:, :, None]
        m0 = m[:, 0:3, :].reshape(RB, 108)
        m1 = m[:, 3:6, :].reshape(RB, 108)
        z = jnp.zeros((RB, 4), F32)
        o_ref[...] = jnp.concatenate([m0, z, m1, z], axis=1)

    return pl.pallas_call(
        body,
        grid=(GE,),
        in_specs=[
            pl.BlockSpec((RB, 48), lambda i: (i, 0)),
            pl.BlockSpec((RB, 8), lambda i: (i, 0)),
            pl.BlockSpec((RB, 8), lambda i: (i, 0)),
        ],
        out_specs=pl.BlockSpec((RB, 224), lambda i: (i, 0)),
        out_shape=jax.ShapeDtypeStruct((E_PAD, 224), F32),
    )(a_g, ex8, rd_g)


def _tc_out12(mp, bias):
    def body(p_ref, b_ref, o_ref):
        s = p_ref[0:NPAD, 0:36] + p_ref[NPAD:2 * NPAD, 0:36] + b_ref[...]
        o_ref[...] = jnp.maximum(s, 0.0)

    return pl.pallas_call(
        body,
        out_shape=jax.ShapeDtypeStruct((NPAD, 36), F32),
    )(mp, bias)


def _tc_final(sp, W3, b3):
    def body(s_ref, w_ref, b_ref, o_ref):
        sv = s_ref[...]
        acc = jnp.zeros((N, 128), F32)
        for hh in range(H):
            half = hh // 3
            off = (hh % 3) * 36
            sh = sv[half * NPAD:half * NPAD + N, off:off + 36]
            acc = acc + jnp.dot(sh, w_ref[...][:, hh * 128:(hh + 1) * 128],
                                preferred_element_type=F32)
        acc = acc * (1.0 / H) + b_ref[...]
        m = jnp.max(acc, axis=0, keepdims=True)
        e = jnp.exp(acc - m)
        o_ref[...] = e / jnp.sum(e, axis=0, keepdims=True)

    return pl.pallas_call(
        body,
        out_shape=jax.ShapeDtypeStruct((N, 128), F32),
    )(sp, W3, b3)


@jax.jit
def kernel(x, edge_index, W1, a1s, a1d, b1, W2, a2s, a2d, b2, W3, a3s, a3d, b3):
    loops = jnp.arange(N, dtype=jnp.int32)
    pad = jnp.full((E_PAD - E_TOT,), N, dtype=jnp.int32)
    src = jnp.concatenate([edge_index[0].astype(jnp.int32), loops, pad])
    dst = jnp.concatenate([edge_index[1].astype(jnp.int32), loops, pad])
    src2 = src.reshape(NW, NB_W, BLK)
    dst2 = dst.reshape(NW, NB_W, BLK)
    dst3 = dst.reshape(NS, NB_C, BLK)

    xp = jnp.pad(x, ((0, NPAD - N), (0, 0)))
    z8 = jnp.zeros((ZR, 8), F32)
    z48 = jnp.zeros((ZR, 48), F32)
    z112 = jnp.zeros((ZR, 112), F32)

    def attention_coefs(A, B):
        a_g = _sc_gather(A, src2, 48)
        b_g = _sc_gather(B, dst2, 8)
        ex8 = _tc_ex(a_g, b_g)
        p = _sc_scatter(ex8, dst2, z8, 8)
        rd = _tc_rdenom(p)
        rd_g = _sc_gather(rd, dst2, 8)
        return a_g, ex8, rd_g

    h = xp
    for (W, a_s, a_d, b, din) in (
        (W1, a1s, a1d, b1, 128),
        (W2, a2s, a2d, b2, 36),
    ):
        A, B = _tc_prep12(h, W, a_s.reshape(H, 6), a_d.reshape(H, 6), din)
        a_g, ex8, rd_g = attention_coefs(A, B)
        msg = _tc_msg12(a_g, ex8, rd_g)
        mp = _sc_scatter(msg, dst2, z48, 48)
        h = _tc_out12(mp, b.reshape(1, 36))

    A, B = _tc_prep3(h, W3, a3s.reshape(H, 128), a3d.reshape(H, 128))
    a_g, ex8, rd_g = attention_coefs(A, B)
    msg = _tc_msg3(a_g, ex8, rd_g)
    sp = _sc_scatter3(msg, dst3, z112)
    return _tc_final(sp, W3, b3.reshape(1, 128))


# speedup vs baseline: 20.4142x; 1.0465x over previous
import jax
import jax.numpy as jnp
from jax import lax
from jax.experimental import pallas as pl
from jax.experimental.pallas import tpu as pltpu
from jax.experimental.pallas import tpu_sc as plsc

N = 10000
NPAD = 10240
E_RAW = 320000
E_TOT = E_RAW + N
BLK = 128
NC, NS = 2, 16
NW = NC * NS
NBT = -(-E_TOT // BLK)
NBT = -(-NBT // NW) * NW
E_PAD = NBT * BLK
NB_W = NBT // NW
NB_C = NBT // NS
ZR = NPAD // NS
H = 6
RB = 4096
GE = E_PAD // RB
F32 = jnp.float32


def _mesh():
    return plsc.VectorSubcoreMesh(core_axis_name="c", subcore_axis_name="s")


_SC_PARAMS = pltpu.CompilerParams(use_tc_tiling_on_sc=False)


NBUF = 3


def _gather_loop(table_hbm, idx_v, out_hbm, base_blk, nblk, rows, gsems, wsem):
    d = rows[0].shape[1]

    def group(g, carry):
        j0 = g * NBUF
        gd = [pltpu.async_copy(table_hbm.at[idx_v.at[j0 + b]], rows[b], gsems[b])
              for b in range(NBUF)]
        wd = []
        for b in range(NBUF):
            gd[b].wait()
            wd.append(pltpu.async_copy(
                rows[b], out_hbm.at[pl.ds((base_blk + j0 + b) * BLK, BLK)], wsem))
        for b in range(NBUF):
            wd[b].wait()
        return carry

    lax.fori_loop(0, nblk // NBUF, group, 0)


def _sc_gather(table, idx2, d):
    def body(table_hbm, idx_hbm, out_hbm, idx_v, r0, r1, r2, s0, s1, s2, wsem):
        wid = lax.axis_index("s") * NC + lax.axis_index("c")
        pltpu.sync_copy(idx_hbm.at[wid], idx_v)
        _gather_loop(table_hbm, idx_v, out_hbm, wid * NB_W, NB_W,
                     [r0, r1, r2], [s0, s1, s2], wsem)

    return pl.kernel(
        body,
        out_type=jax.ShapeDtypeStruct((E_PAD, d), F32),
        mesh=_mesh(),
        scratch_types=[
            pltpu.VMEM((NB_W, BLK), jnp.int32),
            pltpu.VMEM((BLK, d), F32),
            pltpu.VMEM((BLK, d), F32),
            pltpu.VMEM((BLK, d), F32),
            pltpu.SemaphoreType.DMA,
            pltpu.SemaphoreType.DMA,
            pltpu.SemaphoreType.DMA,
            pltpu.SemaphoreType.DMA,
        ],
        compiler_params=_SC_PARAMS,
    )(table, idx2)


def _sc_gather_ab(ta, tb, idx_a, idx_b):
    def body(ta_hbm, tb_hbm, ia_hbm, ib_hbm, oa_hbm, ob_hbm,
             ia_v, ib_v, a0, a1, a2, b0, b1, b2, s0, s1, s2, t0, t1, t2, wsem):
        wid = lax.axis_index("s") * NC + lax.axis_index("c")
        pltpu.sync_copy(ia_hbm.at[wid], ia_v)
        pltpu.sync_copy(ib_hbm.at[wid], ib_v)
        _gather_loop(ta_hbm, ia_v, oa_hbm, wid * NB_W, NB_W,
                     [a0, a1, a2], [s0, s1, s2], wsem)
        _gather_loop(tb_hbm, ib_v, ob_hbm, wid * NB_W, NB_W,
                     [b0, b1, b2], [t0, t1, t2], wsem)

    return pl.kernel(
        body,
        out_type=[
            jax.ShapeDtypeStruct((E_PAD, 48), F32),
            jax.ShapeDtypeStruct((E_PAD, 8), F32),
        ],
        mesh=_mesh(),
        scratch_types=[
            pltpu.VMEM((NB_W, BLK), jnp.int32),
            pltpu.VMEM((NB_W, BLK), jnp.int32),
            pltpu.VMEM((BLK, 48), F32),
            pltpu.VMEM((BLK, 48), F32),
            pltpu.VMEM((BLK, 48), F32),
            pltpu.VMEM((BLK, 8), F32),
            pltpu.VMEM((BLK, 8), F32),
            pltpu.VMEM((BLK, 8), F32),
            pltpu.SemaphoreType.DMA,
            pltpu.SemaphoreType.DMA,
            pltpu.SemaphoreType.DMA,
            pltpu.SemaphoreType.DMA,
            pltpu.SemaphoreType.DMA,
            pltpu.SemaphoreType.DMA,
            pltpu.SemaphoreType.DMA,
        ],
        compiler_params=_SC_PARAMS,
    )(ta, tb, idx_a, idx_b)


def _scatter_loop(vals_hbm, col_off, idx_v, acc, base_blk, nblk, bufs, lsems, asem):
    d = bufs[0].shape[1]

    def group(g, carry):
        j0 = g * NBUF
        if col_off is None:
            ld = [pltpu.async_copy(
                vals_hbm.at[pl.ds((base_blk + j0 + b) * BLK, BLK)],
                bufs[b], lsems[b]) for b in range(NBUF)]
        else:
            ld = [pltpu.async_copy(
                vals_hbm.at[pl.ds((base_blk + j0 + b) * BLK, BLK),
                            pl.ds(col_off, d)],
                bufs[b], lsems[b]) for b in range(NBUF)]
        ad = []
        for b in range(NBUF):
            ld[b].wait()
            ad.append(pltpu.async_copy(bufs[b], acc.at[idx_v.at[j0 + b]],
                                       asem, add=True))
        for b in range(NBUF):
            ad[b].wait()
        return carry

    lax.fori_loop(0, nblk // NBUF, group, 0)


def _sc_scatter(vals, idx2, zrows, d):
    def body(vals_hbm, idx_hbm, z_hbm, out_hbm, idx_v, v0, v1, v2, acc,
             s0, s1, s2, asem):
        cid = lax.axis_index("c")
        sid = lax.axis_index("s")
        wid = sid * NC + cid
        pltpu.sync_copy(z_hbm, acc.at[pl.ds(sid * ZR, ZR)])
        pltpu.sync_copy(idx_hbm.at[wid], idx_v)
        plsc.subcore_barrier()
        _scatter_loop(vals_hbm, None, idx_v, acc, wid * NB_W, NB_W,
                      [v0, v1, v2], [s0, s1, s2], asem)
        plsc.subcore_barrier()
        pltpu.sync_copy(acc.at[pl.ds(sid * ZR, ZR)],
                        out_hbm.at[pl.ds(cid * NPAD + sid * ZR, ZR)])

    return pl.kernel(
        body,
        out_type=jax.ShapeDtypeStruct((NC * NPAD, d), F32),
        mesh=_mesh(),
        scratch_types=[
            pltpu.VMEM((NB_W, BLK), jnp.int32),
            pltpu.VMEM((BLK, d), F32),
            pltpu.VMEM((BLK, d), F32),
            pltpu.VMEM((BLK, d), F32),
            pltpu.VMEM_SHARED((NPAD, d), F32),
            pltpu.SemaphoreType.DMA,
            pltpu.SemaphoreType.DMA,
            pltpu.SemaphoreType.DMA,
            pltpu.SemaphoreType.DMA,
        ],
        compiler_params=_SC_PARAMS,
    )(vals, idx2, zrows)


def _sc_scatter3(vals, idx2, zrows):
    d = 112

    def body(vals_hbm, idx_hbm, z_hbm, out_hbm, idx_v, v0, v1, v2, acc,
             s0, s1, s2, asem):
        cid = lax.axis_index("c")
        sid = lax.axis_index("s")
        pltpu.sync_copy(z_hbm, acc.at[pl.ds(sid * ZR, ZR)])
        pltpu.sync_copy(idx_hbm.at[sid], idx_v)
        plsc.subcore_barrier()

        def step(j, carry):
            pltpu.sync_copy(
                vals_hbm.at[pl.ds((sid * NB_C + j) * BLK, BLK), pl.ds(cid * d, d)],
                v0)
            pltpu.sync_copy(v0, acc.at[idx_v.at[j]], add=True)
            return carry

        lax.fori_loop(0, NB_C, step, 0)
        plsc.subcore_barrier()
        pltpu.sync_copy(acc.at[pl.ds(sid * ZR, ZR)],
                        out_hbm.at[pl.ds(cid * NPAD + sid * ZR, ZR)])

    return pl.kernel(
        body,
        out_type=jax.ShapeDtypeStruct((NC * NPAD, d), F32),
        mesh=_mesh(),
        scratch_types=[
            pltpu.VMEM((NB_C, BLK), jnp.int32),
            pltpu.VMEM((BLK, d), F32),
            pltpu.VMEM((BLK, d), F32),
            pltpu.VMEM((BLK, d), F32),
            pltpu.VMEM_SHARED((NPAD, d), F32),
            pltpu.SemaphoreType.DMA,
            pltpu.SemaphoreType.DMA,
            pltpu.SemaphoreType.DMA,
            pltpu.SemaphoreType.DMA,
        ],
        compiler_params=_SC_PARAMS,
    )(vals, idx2, zrows)


_R = 2560


def _tc_prep12(h, W, a_s, a_d, din):
    def body(h_ref, w_ref, as_ref, ad_ref, a_ref, b_ref):
        xw = jnp.dot(h_ref[...], w_ref[...], preferred_element_type=F32)
        x3 = xw.reshape(_R, H, 6)
        asr = jnp.sum(x3 * as_ref[...][None], axis=-1)
        ads = jnp.sum(x3 * ad_ref[...][None], axis=-1)
        z = jnp.zeros((_R, 6), F32)
        a_ref[...] = jnp.concatenate([xw, asr, z], axis=1)
        b_ref[...] = jnp.concatenate([ads, jnp.zeros((_R, 2), F32)], axis=1)

    return pl.pallas_call(
        body,
        grid=(NPAD // _R,),
        in_specs=[
            pl.BlockSpec((_R, din), lambda i: (i, 0)),
            pl.BlockSpec((din, H * 6), lambda i: (0, 0)),
            pl.BlockSpec((H, 6), lambda i: (0, 0)),
            pl.BlockSpec((H, 6), lambda i: (0, 0)),
        ],
        out_specs=[
            pl.BlockSpec((_R, 48), lambda i: (i, 0)),
            pl.BlockSpec((_R, 8), lambda i: (i, 0)),
        ],
        out_shape=[
            jax.ShapeDtypeStruct((NPAD, 48), F32),
            jax.ShapeDtypeStruct((NPAD, 8), F32),
        ],
    )(h, W, a_s, a_d)


def _tc_prep3(h, W, a_s, a_d):
    def body(h_ref, w_ref, as_ref, ad_ref, a_ref, b_ref):
        hv = h_ref[...]
        xw = jnp.dot(hv, w_ref[...], preferred_element_type=F32)
        x3 = xw.reshape(_R, H, 128)
        asr = jnp.sum(x3 * as_ref[...][None], axis=-1)
        ads = jnp.sum(x3 * ad_ref[...][None], axis=-1)
        z = jnp.zeros((_R, 6), F32)
        a_ref[...] = jnp.concatenate([hv, asr, z], axis=1)
        b_ref[...] = jnp.concatenate([ads, jnp.zeros((_R, 2), F32)], axis=1)

    return pl.pallas_call(
        body,
        grid=(NPAD // _R,),
        in_specs=[
            pl.BlockSpec((_R, 36), lambda i: (i, 0)),
            pl.BlockSpec((36, H * 128), lambda i: (0, 0)),
            pl.BlockSpec((H, 128), lambda i: (0, 0)),
            pl.BlockSpec((H, 128), lambda i: (0, 0)),
        ],
        out_specs=[
            pl.BlockSpec((_R, 48), lambda i: (i, 0)),
            pl.BlockSpec((_R, 8), lambda i: (i, 0)),
        ],
        out_shape=[
            jax.ShapeDtypeStruct((NPAD, 48), F32),
            jax.ShapeDtypeStruct((NPAD, 8), F32),
        ],
    )(h, W, a_s, a_d)


def _tc_ex(a_g, b_g):
    def body(a_ref, b_ref, o_ref):
        al = a_ref[...][:, 36:42] + b_ref[...][:, 0:6]
        al = jnp.maximum(al, 0.2 * al)
        o_ref[...] = jnp.concatenate(
            [jnp.exp(al), jnp.zeros((RB, 2), F32)], axis=1)

    return pl.pallas_call(
        body,
        grid=(GE,),
        in_specs=[
            pl.BlockSpec((RB, 48), lambda i: (i, 0)),
            pl.BlockSpec((RB, 8), lambda i: (i, 0)),
        ],
        out_specs=pl.BlockSpec((RB, 8), lambda i: (i, 0)),
        out_shape=jax.ShapeDtypeStruct((E_PAD, 8), F32),
    )(a_g, b_g)


def _tc_rdenom(p):
    def body(p_ref, o_ref):
        d = p_ref[0:NPAD, :] + p_ref[NPAD:2 * NPAD, :]
        o_ref[...] = 1.0 / (d + 1e-16)

    return pl.pallas_call(
        body,
        out_shape=jax.ShapeDtypeStruct((NPAD, 8), F32),
    )(p)


def _tc_msg12(a_g, ex8, rd_g):
    def body(a_ref, e_ref, r_ref, o_ref):
        coef = e_ref[...][:, 0:6] * r_ref[...][:, 0:6]
        xws = a_ref[...][:, 0:36].reshape(RB, H, 6)
        m = (xws * coef[:, :, None]).reshape(RB, 36)
        o_ref[...] = jnp.concatenate([m, jnp.zeros((RB, 12), F32)], axis=1)

    return pl.pallas_call(
        body,
        grid=(GE,),
        in_specs=[
            pl.BlockSpec((RB, 48), lambda i: (i, 0)),
            pl.BlockSpec((RB, 8), lambda i: (i, 0)),
            pl.BlockSpec((RB, 8), lambda i: (i, 0)),
        ],
        out_specs=pl.BlockSpec((RB, 48), lambda i: (i, 0)),
        out_shape=jax.ShapeDtypeStruct((E_PAD, 48), F32),
    )(a_g, ex8, rd_g)


def _tc_msg3(a_g, ex8, rd_g):
    def body(a_ref, e_ref, r_ref, o_ref):
        coef = e_ref[...][:, 0:6] * r_ref[...][:, 0:6]
        h2s = a_ref[...][:, 0:36]
        m = h2s[:, None, :] * coef[:, :, None]
        m0 = m[:, 0:3, :].reshape(RB, 108)
        m1 = m[:, 3:6, :].reshape(RB, 108)
        z = jnp.zeros((RB, 4), F32)
        o_ref[...] = jnp.concatenate([m0, z, m1, z], axis=1)

    return pl.pallas_call(
        body,
        grid=(GE,),
        in_specs=[
            pl.BlockSpec((RB, 48), lambda i: (i, 0)),
            pl.BlockSpec((RB, 8), lambda i: (i, 0)),
            pl.BlockSpec((RB, 8), lambda i: (i, 0)),
        ],
        out_specs=pl.BlockSpec((RB, 224), lambda i: (i, 0)),
        out_shape=jax.ShapeDtypeStruct((E_PAD, 224), F32),
    )(a_g, ex8, rd_g)


def _tc_out12(mp, bias):
    def body(p_ref, b_ref, o_ref):
        s = p_ref[0:NPAD, 0:36] + p_ref[NPAD:2 * NPAD, 0:36] + b_ref[...]
        o_ref[...] = jnp.maximum(s, 0.0)

    return pl.pallas_call(
        body,
        out_shape=jax.ShapeDtypeStruct((NPAD, 36), F32),
    )(mp, bias)


def _tc_final(sp, W3, b3):
    def body(s_ref, w_ref, b_ref, o_ref):
        sv = s_ref[...]
        acc = jnp.zeros((N, 128), F32)
        for hh in range(H):
            half = hh // 3
            off = (hh % 3) * 36
            sh = sv[half * NPAD:half * NPAD + N, off:off + 36]
            acc = acc + jnp.dot(sh, w_ref[...][:, hh * 128:(hh + 1) * 128],
                                preferred_element_type=F32)
        acc = acc * (1.0 / H) + b_ref[...]
        m = jnp.max(acc, axis=0, keepdims=True)
        e = jnp.exp(acc - m)
        o_ref[...] = e / jnp.sum(e, axis=0, keepdims=True)

    return pl.pallas_call(
        body,
        out_shape=jax.ShapeDtypeStruct((N, 128), F32),
    )(sp, W3, b3)


@jax.jit
def kernel(x, edge_index, W1, a1s, a1d, b1, W2, a2s, a2d, b2, W3, a3s, a3d, b3):
    loops = jnp.arange(N, dtype=jnp.int32)
    pad = jnp.full((E_PAD - E_TOT,), N, dtype=jnp.int32)
    src = jnp.concatenate([edge_index[0].astype(jnp.int32), loops, pad])
    dst = jnp.concatenate([edge_index[1].astype(jnp.int32), loops, pad])
    src2 = src.reshape(NW, NB_W, BLK)
    dst2 = dst.reshape(NW, NB_W, BLK)
    dst3 = dst.reshape(NS, NB_C, BLK)

    xp = jnp.pad(x, ((0, NPAD - N), (0, 0)))
    z8 = jnp.zeros((ZR, 8), F32)
    z48 = jnp.zeros((ZR, 48), F32)
    z112 = jnp.zeros((ZR, 112), F32)

    def attention_coefs(A, B):
        a_g, b_g = _sc_gather_ab(A, B, src2, dst2)
        ex8 = _tc_ex(a_g, b_g)
        p = _sc_scatter(ex8, dst2, z8, 8)
        rd = _tc_rdenom(p)
        rd_g = _sc_gather(rd, dst2, 8)
        return a_g, ex8, rd_g

    h = xp
    for (W, a_s, a_d, b, din) in (
        (W1, a1s, a1d, b1, 128),
        (W2, a2s, a2d, b2, 36),
    ):
        A, B = _tc_prep12(h, W, a_s.reshape(H, 6), a_d.reshape(H, 6), din)
        a_g, ex8, rd_g = attention_coefs(A, B)
        msg = _tc_msg12(a_g, ex8, rd_g)
        mp = _sc_scatter(msg, dst2, z48, 48)
        h = _tc_out12(mp, b.reshape(1, 36))

    A, B = _tc_prep3(h, W3, a3s.reshape(H, 128), a3d.reshape(H, 128))
    a_g, ex8, rd_g = attention_coefs(A, B)
    msg = _tc_msg3(a_g, ex8, rd_g)
    sp = _sc_scatter3(msg, dst3, z112)
    return _tc_final(sp, W3, b3.reshape(1, 128))
